# baseline (speedup 1.0000x reference)
import functools

import jax
import jax.numpy as jnp
from jax import lax
from jax.experimental import pallas as pl
from jax.experimental.pallas import tpu as pltpu
from jax.experimental.pallas import tpu_sc as plsc

N = 10000
NP = 10240
E = 160000
EMB = 128
CHUNK = 128
ROWS = E // CHUNK
NC = 2
NS = 16
NW = NC * NS
NPT = NP // NS
EBLK = 2000

DDE_BASE = ROWS // NS
DDE_EXTRA = ROWS - NS * DDE_BASE

_SC_PARAMS = pltpu.CompilerParams(use_tc_tiling_on_sc=False)


def _dde_body(ei_ref, topic4_ref, zeros16_ref, pe_ref,
              gidx_all, aidx_all, msg2, node_v, acc_sh, sem0, sem1):
    cid = lax.axis_index("c")
    sid = lax.axis_index("s")
    iota = lax.iota(jnp.int32, 16)
    mask01 = iota < 2
    constrow = jnp.where(iota == 2, 1.0, 0.0)

    start = (sid * DDE_BASE + jnp.minimum(sid, DDE_EXTRA)) * CHUNK
    cnt = DDE_BASE + jnp.where(sid < DDE_EXTRA, 1, 0)
    sems = (sem0, sem1)

    def chain(c):
        pltpu.sync_copy(ei_ref.at[c, pl.ds(start, DDE_BASE * CHUNK)],
                        gidx_all.at[pl.ds(0, DDE_BASE * CHUNK)])
        pltpu.sync_copy(ei_ref.at[1 - c, pl.ds(start, DDE_BASE * CHUNK)],
                        aidx_all.at[pl.ds(0, DDE_BASE * CHUNK)])

        @pl.when(sid < DDE_EXTRA)
        def _():
            pltpu.sync_copy(
                ei_ref.at[c, pl.ds(start + DDE_BASE * CHUNK, CHUNK)],
                gidx_all.at[pl.ds(DDE_BASE * CHUNK, CHUNK)])
            pltpu.sync_copy(
                ei_ref.at[1 - c, pl.ds(start + DDE_BASE * CHUNK, CHUNK)],
                aidx_all.at[pl.ds(DDE_BASE * CHUNK, CHUNK)])

        for r in range(2):
            pltpu.sync_copy(zeros16_ref.at[pl.ds(sid * NPT, NPT)],
                            acc_sh.at[pl.ds(sid * NPT, NPT)])
            plsc.subcore_barrier()

            table = topic4_ref if r == 0 else pe_ref.at[c, 0]

            def issue(j, s):
                pltpu.async_copy(
                    table.at[gidx_all.at[pl.ds(j * CHUNK, CHUNK)]],
                    msg2.at[s], sems[s])

            def drain(s):
                pltpu.make_async_copy(table.at[pl.ds(0, CHUNK)],
                                      msg2.at[s], sems[s]).wait()

            def scatter(j, s):
                pltpu.sync_copy(
                    msg2.at[s],
                    acc_sh.at[aidx_all.at[pl.ds(j * CHUNK, CHUNK)]],
                    add=True)

            issue(0, 0)
            issue(1, 1)

            def pair(j2, carry):
                a = 2 * j2
                drain(0)
                scatter(a, 0)

                @pl.when(a + 2 < cnt)
                def _():
                    issue(a + 2, 0)
                drain(1)
                scatter(a + 1, 1)

                @pl.when(a + 3 < cnt)
                def _():
                    issue(a + 3, 1)
                return carry
            lax.fori_loop(0, DDE_BASE // 2, pair, 0)

            @pl.when(sid < DDE_EXTRA)
            def _():
                drain(0)
                scatter(DDE_BASE, 0)

            plsc.subcore_barrier()

            pltpu.sync_copy(acc_sh.at[pl.ds(sid * NPT, NPT)], node_v)

            def dbody(i, carry):
                row = node_v[i, :]
                dvec = jnp.broadcast_to(row[2], (16,))
                inv = 1.0 / jnp.maximum(dvec, 1.0)
                node_v[i, :] = jnp.where(mask01, row * inv, constrow)
                return carry
            lax.fori_loop(0, NPT, dbody, 0)

            pltpu.sync_copy(node_v, pe_ref.at[c, r, pl.ds(sid * NPT, NPT)])
            plsc.subcore_barrier()

    @pl.when(cid == 0)
    def _():
        chain(0)

    @pl.when(cid == 1)
    def _():
        chain(1)


_dde = pl.kernel(
    _dde_body,
    out_type=jax.ShapeDtypeStruct((2, 2, NP, 16), jnp.float32),
    mesh=plsc.VectorSubcoreMesh(core_axis_name="c", subcore_axis_name="s"),
    scratch_types=[
        pltpu.VMEM(((DDE_BASE + 2) * CHUNK,), jnp.int32),
        pltpu.VMEM(((DDE_BASE + 2) * CHUNK,), jnp.int32),
        pltpu.VMEM((2, CHUNK, 16), jnp.float32),
        pltpu.VMEM((NPT, 16), jnp.float32),
        pltpu.VMEM_SHARED((NP, 16), jnp.float32),
        pltpu.SemaphoreType.DMA,
        pltpu.SemaphoreType.DMA,
    ],
    compiler_params=_SC_PARAMS,
)


def _make_gather(row_lo, nrows):
    base = nrows // NW
    extra = nrows - NW * base

    def body(ei_ref, zs_ref, zd_ref, g_ref,
             sidx_all, didx_all, abuf, bbuf, obuf,
             sa0, sa1, sb0, sb1, sw0, sw1):
        cid = lax.axis_index("c")
        sid = lax.axis_index("s")
        wid = sid * NC + cid
        rel_start = wid * base + jnp.minimum(wid, extra)
        start = (row_lo + rel_start) * CHUNK
        out_start = rel_start * CHUNK
        cnt = base + jnp.where(wid < extra, 1, 0)
        sas = (sa0, sa1)
        sbs = (sb0, sb1)
        sws = (sw0, sw1)

        pltpu.sync_copy(ei_ref.at[0, pl.ds(start, base * CHUNK)],
                        sidx_all.at[pl.ds(0, base * CHUNK)])
        pltpu.sync_copy(ei_ref.at[1, pl.ds(start, base * CHUNK)],
                        didx_all.at[pl.ds(0, base * CHUNK)])

        @pl.when(wid < extra)
        def _():
            pltpu.sync_copy(ei_ref.at[0, pl.ds(start + base * CHUNK, CHUNK)],
                            sidx_all.at[pl.ds(base * CHUNK, CHUNK)])
            pltpu.sync_copy(ei_ref.at[1, pl.ds(start + base * CHUNK, CHUNK)],
                            didx_all.at[pl.ds(base * CHUNK, CHUNK)])

        def issue(j, s):
            pltpu.async_copy(
                zs_ref.at[sidx_all.at[pl.ds(j * CHUNK, CHUNK)]],
                abuf.at[s], sas[s])
            pltpu.async_copy(
                zd_ref.at[didx_all.at[pl.ds(j * CHUNK, CHUNK)]],
                bbuf.at[s], sbs[s])

        def drain_g(s):
            pltpu.make_async_copy(zs_ref.at[pl.ds(0, CHUNK)], abuf.at[s],
                                  sas[s]).wait()
            pltpu.make_async_copy(zd_ref.at[pl.ds(0, CHUNK)], bbuf.at[s],
                                  sbs[s]).wait()

        def drain_w(s):
            pltpu.make_async_copy(obuf.at[s], g_ref.at[pl.ds(0, CHUNK)],
                                  sws[s]).wait()

        def process(j, s):
            drain_g(s)

            @pl.when(j >= 2)
            def _():
                drain_w(s)

            def add_row(i, c2):
                for k in range(EMB // 16):
                    sl = pl.ds(k * 16, 16)
                    obuf[s, i, sl] = abuf[s, i, sl] + bbuf[s, i, sl]
                return c2
            lax.fori_loop(0, CHUNK, add_row, 0)
            pltpu.async_copy(obuf.at[s],
                             g_ref.at[pl.ds(out_start + j * CHUNK, CHUNK)],
                             sws[s])

            @pl.when(j + 2 < cnt)
            def _():
                issue(j + 2, s)

        issue(0, 0)
        issue(1, 1)

        def pair(j2, carry):
            process(2 * j2, 0)
            process(2 * j2 + 1, 1)
            return carry
        lax.fori_loop(0, base // 2, pair, 0)

        if base % 2 == 1:
            process(base - 1, 0)

            @pl.when(wid < extra)
            def _():
                process(base, 1)
        else:
            @pl.when(wid < extra)
            def _():
                process(base, 0)

        drain_w(0)
        drain_w(1)

    return pl.kernel(
        body,
        out_type=jax.ShapeDtypeStruct((nrows * CHUNK, EMB), jnp.float32),
        mesh=plsc.VectorSubcoreMesh(core_axis_name="c", subcore_axis_name="s"),
        scratch_types=[
            pltpu.VMEM(((base + 3) * CHUNK,), jnp.int32),
            pltpu.VMEM(((base + 3) * CHUNK,), jnp.int32),
            pltpu.VMEM((2, CHUNK, EMB), jnp.float32),
            pltpu.VMEM((2, CHUNK, EMB), jnp.float32),
            pltpu.VMEM((2, CHUNK, EMB), jnp.float32),
            pltpu.SemaphoreType.DMA,
            pltpu.SemaphoreType.DMA,
            pltpu.SemaphoreType.DMA,
            pltpu.SemaphoreType.DMA,
            pltpu.SemaphoreType.DMA,
            pltpu.SemaphoreType.DMA,
        ],
    )


ROWS_H = ROWS // 2
_gather0 = _make_gather(0, ROWS_H)
_gather1 = _make_gather(ROWS_H, ROWS - ROWS_H)


def _ztables_kernel(ent_ref, topic_ref, pe_ref, q_ref, wq_ref,
                    ws_ref, wd_ref, wsml_ref, b1_ref, zs_ref, zd_ref):
    qc = jnp.dot(q_ref[...], wq_ref[...],
                 preferred_element_type=jnp.float32) + b1_ref[...]
    ent = ent_ref[...]
    zs = jnp.dot(ent, ws_ref[...], preferred_element_type=jnp.float32) + qc
    zd = jnp.dot(ent, wd_ref[...], preferred_element_type=jnp.float32)

    feats = [topic_ref[:, 0:1], topic_ref[:, 1:2]]
    for c in range(2):
        for r in range(2):
            pe = pe_ref[c, r]
            feats.append(pe[:, 0:1])
            feats.append(pe[:, 1:2])
    for k, f in enumerate(feats):
        zs = zs + f * wsml_ref[k, :][None, :]
        zd = zd + f * wsml_ref[10 + k, :][None, :]
    zs_ref[...] = zs
    zd_ref[...] = zd


def _ztables(entity_embs, topic, pe, q_emb, Wq, Ws, Wd, Wsml, b1row):
    NB = 1000
    return pl.pallas_call(
        _ztables_kernel,
        grid=(N // NB,),
        in_specs=[
            pl.BlockSpec((NB, EMB), lambda i: (i, 0)),
            pl.BlockSpec((NB, 2), lambda i: (i, 0)),
            pl.BlockSpec((2, 2, NB, 16), lambda i: (0, 0, i, 0)),
            pl.BlockSpec((1, EMB), lambda i: (0, 0)),
            pl.BlockSpec((EMB, EMB), lambda i: (0, 0)),
            pl.BlockSpec((EMB, EMB), lambda i: (0, 0)),
            pl.BlockSpec((EMB, EMB), lambda i: (0, 0)),
            pl.BlockSpec((20, EMB), lambda i: (0, 0)),
            pl.BlockSpec((1, EMB), lambda i: (0, 0)),
        ],
        out_specs=(
            pl.BlockSpec((NB, EMB), lambda i: (i, 0)),
            pl.BlockSpec((NB, EMB), lambda i: (i, 0)),
        ),
        out_shape=(
            jax.ShapeDtypeStruct((N, EMB), jnp.float32),
            jax.ShapeDtypeStruct((N, EMB), jnp.float32),
        ),
    )(entity_embs, topic, pe[:, :, :N], q_emb, Wq, Ws, Wd, Wsml, b1row)


def _edge_mlp_kernel(rel_ref, g_ref, wr_ref, w2_ref, b2_ref, out_ref):
    z = jnp.dot(rel_ref[...], wr_ref[...],
                preferred_element_type=jnp.float32) + g_ref[...]
    h = jnp.maximum(z, 0.0)
    out_ref[...] = jnp.dot(h, w2_ref[...],
                           preferred_element_type=jnp.float32) + b2_ref[...]


def _edge_mlp(relation_embs, G, Wr, W2, b2row, blk_off, nblk):
    ne = G.shape[0]
    return pl.pallas_call(
        _edge_mlp_kernel,
        grid=(nblk,),
        in_specs=[
            pl.BlockSpec((EBLK, EMB), lambda i: (i + blk_off, 0)),
            pl.BlockSpec((EBLK, EMB), lambda i: (i, 0)),
            pl.BlockSpec((EMB, EMB), lambda i: (0, 0)),
            pl.BlockSpec((EMB, 1), lambda i: (0, 0)),
            pl.BlockSpec((1, 1), lambda i: (0, 0)),
        ],
        out_specs=pl.BlockSpec((EBLK, 1), lambda i: (i, 0)),
        out_shape=jax.ShapeDtypeStruct((ne, 1), jnp.float32),
    )(relation_embs, G, Wr, W2, b2row)


def kernel(edge_index, q_emb, entity_embs, relation_embs,
           topic_entity_one_hot, W1, b1, W2, b2):
    ei = edge_index.astype(jnp.int32)
    topic4 = jnp.concatenate(
        [topic_entity_one_hot,
         jnp.ones((N, 1), jnp.float32),
         jnp.zeros((N, 13), jnp.float32)], axis=1)
    topic4 = jnp.pad(topic4, ((0, NP - N), (0, 0)))
    zeros16 = jnp.zeros((NP, 16), jnp.float32)

    pe = _dde(ei, topic4, zeros16)

    Wsml = jnp.concatenate([W1[256:266], W1[522:532]], axis=0)
    Zs, Zd = _ztables(entity_embs, topic_entity_one_hot, pe, q_emb,
                      W1[0:128], W1[128:256], W1[394:522], Wsml,
                      b1.reshape(1, EMB))

    EH = ROWS_H * CHUNK
    G0 = _gather0(ei, Zs, Zd)
    G1 = _gather1(ei, Zs, Zd)

    Wr = W1[266:394]
    b2row = b2.reshape(1, 1)
    nblk_h = EH // EBLK
    out0 = _edge_mlp(relation_embs, G0, Wr, W2, b2row, 0, nblk_h)
    out1 = _edge_mlp(relation_embs, G1, Wr, W2, b2row, nblk_h, nblk_h)
    return jnp.concatenate([out0, out1], axis=0)


# speedup vs baseline: 1.1583x; 1.1583x over previous
import functools

import jax
import jax.numpy as jnp
from jax import lax
from jax.experimental import pallas as pl
from jax.experimental.pallas import tpu as pltpu
from jax.experimental.pallas import tpu_sc as plsc

N = 10000
NP = 10240
E = 160000
EMB = 128
CHUNK = 128
ROWS = E // CHUNK
NC = 2
NS = 16
NW = NC * NS
NPT = NP // NS
EBLK = 3200

DDE_BASE = ROWS // NS
DDE_EXTRA = ROWS - NS * DDE_BASE

_SC_PARAMS = pltpu.CompilerParams(use_tc_tiling_on_sc=False)


def _dde_body(ei_ref, topic4_ref, zeros16_ref, pe_ref,
              gidx_all, aidx_all, msg2, node_v, acc_sh, sem0, sem1):
    cid = lax.axis_index("c")
    sid = lax.axis_index("s")
    iota = lax.iota(jnp.int32, 16)
    mask01 = iota < 2
    constrow = jnp.where(iota == 2, 1.0, 0.0)

    start = (sid * DDE_BASE + jnp.minimum(sid, DDE_EXTRA)) * CHUNK
    cnt = DDE_BASE + jnp.where(sid < DDE_EXTRA, 1, 0)
    sems = (sem0, sem1)

    def chain(c):
        pltpu.sync_copy(ei_ref.at[c, pl.ds(start, DDE_BASE * CHUNK)],
                        gidx_all.at[pl.ds(0, DDE_BASE * CHUNK)])
        pltpu.sync_copy(ei_ref.at[1 - c, pl.ds(start, DDE_BASE * CHUNK)],
                        aidx_all.at[pl.ds(0, DDE_BASE * CHUNK)])

        @pl.when(sid < DDE_EXTRA)
        def _():
            pltpu.sync_copy(
                ei_ref.at[c, pl.ds(start + DDE_BASE * CHUNK, CHUNK)],
                gidx_all.at[pl.ds(DDE_BASE * CHUNK, CHUNK)])
            pltpu.sync_copy(
                ei_ref.at[1 - c, pl.ds(start + DDE_BASE * CHUNK, CHUNK)],
                aidx_all.at[pl.ds(DDE_BASE * CHUNK, CHUNK)])

        for r in range(2):
            pltpu.sync_copy(zeros16_ref.at[pl.ds(sid * NPT, NPT)],
                            acc_sh.at[pl.ds(sid * NPT, NPT)])
            plsc.subcore_barrier()

            table = topic4_ref if r == 0 else pe_ref.at[c, 0]

            def issue(j, s):
                pltpu.async_copy(
                    table.at[gidx_all.at[pl.ds(j * CHUNK, CHUNK)]],
                    msg2.at[s], sems[s])

            def drain(s):
                pltpu.make_async_copy(table.at[pl.ds(0, CHUNK)],
                                      msg2.at[s], sems[s]).wait()

            def scatter(j, s):
                pltpu.sync_copy(
                    msg2.at[s],
                    acc_sh.at[aidx_all.at[pl.ds(j * CHUNK, CHUNK)]],
                    add=True)

            issue(0, 0)
            issue(1, 1)

            def pair(j2, carry):
                a = 2 * j2
                drain(0)
                scatter(a, 0)

                @pl.when(a + 2 < cnt)
                def _():
                    issue(a + 2, 0)
                drain(1)
                scatter(a + 1, 1)

                @pl.when(a + 3 < cnt)
                def _():
                    issue(a + 3, 1)
                return carry
            lax.fori_loop(0, DDE_BASE // 2, pair, 0)

            @pl.when(sid < DDE_EXTRA)
            def _():
                drain(0)
                scatter(DDE_BASE, 0)

            plsc.subcore_barrier()

            pltpu.sync_copy(acc_sh.at[pl.ds(sid * NPT, NPT)], node_v)

            def dbody(i, carry):
                row = node_v[i, :]
                dvec = jnp.broadcast_to(row[2], (16,))
                inv = 1.0 / jnp.maximum(dvec, 1.0)
                node_v[i, :] = jnp.where(mask01, row * inv, constrow)
                return carry
            lax.fori_loop(0, NPT, dbody, 0)

            pltpu.sync_copy(node_v, pe_ref.at[c, r, pl.ds(sid * NPT, NPT)])
            plsc.subcore_barrier()

    @pl.when(cid == 0)
    def _():
        chain(0)

    @pl.when(cid == 1)
    def _():
        chain(1)


_dde = pl.kernel(
    _dde_body,
    out_type=jax.ShapeDtypeStruct((2, 2, NP, 16), jnp.float32),
    mesh=plsc.VectorSubcoreMesh(core_axis_name="c", subcore_axis_name="s"),
    scratch_types=[
        pltpu.VMEM(((DDE_BASE + 2) * CHUNK,), jnp.int32),
        pltpu.VMEM(((DDE_BASE + 2) * CHUNK,), jnp.int32),
        pltpu.VMEM((2, CHUNK, 16), jnp.float32),
        pltpu.VMEM((NPT, 16), jnp.float32),
        pltpu.VMEM_SHARED((NP, 16), jnp.float32),
        pltpu.SemaphoreType.DMA,
        pltpu.SemaphoreType.DMA,
    ],
    compiler_params=_SC_PARAMS,
)


def _make_gather(row_lo, nrows):
    base = nrows // NW
    extra = nrows - NW * base

    def body(ei_ref, zs_ref, zd_ref, g_ref,
             sidx_all, didx_all, abuf, bbuf, obuf,
             sa0, sa1, sb0, sb1, sw0, sw1):
        cid = lax.axis_index("c")
        sid = lax.axis_index("s")
        wid = sid * NC + cid
        rel_start = wid * base + jnp.minimum(wid, extra)
        start = (row_lo + rel_start) * CHUNK
        out_start = rel_start * CHUNK
        cnt = base + jnp.where(wid < extra, 1, 0)
        sas = (sa0, sa1)
        sbs = (sb0, sb1)
        sws = (sw0, sw1)

        pltpu.sync_copy(ei_ref.at[0, pl.ds(start, base * CHUNK)],
                        sidx_all.at[pl.ds(0, base * CHUNK)])
        pltpu.sync_copy(ei_ref.at[1, pl.ds(start, base * CHUNK)],
                        didx_all.at[pl.ds(0, base * CHUNK)])

        @pl.when(wid < extra)
        def _():
            pltpu.sync_copy(ei_ref.at[0, pl.ds(start + base * CHUNK, CHUNK)],
                            sidx_all.at[pl.ds(base * CHUNK, CHUNK)])
            pltpu.sync_copy(ei_ref.at[1, pl.ds(start + base * CHUNK, CHUNK)],
                            didx_all.at[pl.ds(base * CHUNK, CHUNK)])

        def issue(j, s):
            pltpu.async_copy(
                zs_ref.at[sidx_all.at[pl.ds(j * CHUNK, CHUNK)]],
                abuf.at[s], sas[s])
            pltpu.async_copy(
                zd_ref.at[didx_all.at[pl.ds(j * CHUNK, CHUNK)]],
                bbuf.at[s], sbs[s])

        def drain_g(s):
            pltpu.make_async_copy(zs_ref.at[pl.ds(0, CHUNK)], abuf.at[s],
                                  sas[s]).wait()
            pltpu.make_async_copy(zd_ref.at[pl.ds(0, CHUNK)], bbuf.at[s],
                                  sbs[s]).wait()

        def drain_w(s):
            pltpu.make_async_copy(obuf.at[s], g_ref.at[pl.ds(0, CHUNK)],
                                  sws[s]).wait()

        def process(j, s):
            drain_g(s)

            @pl.when(j >= 2)
            def _():
                drain_w(s)

            def add_row(i, c2):
                for k in range(EMB // 16):
                    sl = pl.ds(k * 16, 16)
                    obuf[s, i, sl] = abuf[s, i, sl] + bbuf[s, i, sl]
                return c2
            lax.fori_loop(0, CHUNK, add_row, 0)
            pltpu.async_copy(obuf.at[s],
                             g_ref.at[pl.ds(out_start + j * CHUNK, CHUNK)],
                             sws[s])

            @pl.when(j + 2 < cnt)
            def _():
                issue(j + 2, s)

        issue(0, 0)
        issue(1, 1)

        def pair(j2, carry):
            process(2 * j2, 0)
            process(2 * j2 + 1, 1)
            return carry
        lax.fori_loop(0, base // 2, pair, 0)

        if base % 2 == 1:
            process(base - 1, 0)

            @pl.when(wid < extra)
            def _():
                process(base, 1)
        else:
            @pl.when(wid < extra)
            def _():
                process(base, 0)

        drain_w(0)
        drain_w(1)

    return pl.kernel(
        body,
        out_type=jax.ShapeDtypeStruct((nrows * CHUNK, EMB), jnp.float32),
        mesh=plsc.VectorSubcoreMesh(core_axis_name="c", subcore_axis_name="s"),
        scratch_types=[
            pltpu.VMEM(((base + 3) * CHUNK,), jnp.int32),
            pltpu.VMEM(((base + 3) * CHUNK,), jnp.int32),
            pltpu.VMEM((2, CHUNK, EMB), jnp.float32),
            pltpu.VMEM((2, CHUNK, EMB), jnp.float32),
            pltpu.VMEM((2, CHUNK, EMB), jnp.float32),
            pltpu.SemaphoreType.DMA,
            pltpu.SemaphoreType.DMA,
            pltpu.SemaphoreType.DMA,
            pltpu.SemaphoreType.DMA,
            pltpu.SemaphoreType.DMA,
            pltpu.SemaphoreType.DMA,
        ],
    )


ROWS_H = ROWS // 2
_gather0 = _make_gather(0, ROWS_H)
_gather1 = _make_gather(ROWS_H, ROWS - ROWS_H)


def _ztables_kernel(ent_ref, topic_ref, pe_ref, q_ref, wq_ref,
                    ws_ref, wd_ref, wsml_ref, b1_ref, zs_ref, zd_ref):
    qc = jnp.dot(q_ref[...], wq_ref[...],
                 preferred_element_type=jnp.float32) + b1_ref[...]
    ent = ent_ref[...]
    zs = jnp.dot(ent, ws_ref[...], preferred_element_type=jnp.float32) + qc
    zd = jnp.dot(ent, wd_ref[...], preferred_element_type=jnp.float32)

    feats = [topic_ref[:, 0:1], topic_ref[:, 1:2]]
    for c in range(2):
        for r in range(2):
            pe = pe_ref[c, r]
            feats.append(pe[:, 0:1])
            feats.append(pe[:, 1:2])
    for k, f in enumerate(feats):
        zs = zs + f * wsml_ref[k, :][None, :]
        zd = zd + f * wsml_ref[10 + k, :][None, :]
    zs_ref[...] = zs
    zd_ref[...] = zd


def _ztables(entity_embs, topic, pe, q_emb, Wq, Ws, Wd, Wsml, b1row):
    NB = 1000
    return pl.pallas_call(
        _ztables_kernel,
        grid=(N // NB,),
        in_specs=[
            pl.BlockSpec((NB, EMB), lambda i: (i, 0)),
            pl.BlockSpec((NB, 2), lambda i: (i, 0)),
            pl.BlockSpec((2, 2, NB, 16), lambda i: (0, 0, i, 0)),
            pl.BlockSpec((1, EMB), lambda i: (0, 0)),
            pl.BlockSpec((EMB, EMB), lambda i: (0, 0)),
            pl.BlockSpec((EMB, EMB), lambda i: (0, 0)),
            pl.BlockSpec((EMB, EMB), lambda i: (0, 0)),
            pl.BlockSpec((20, EMB), lambda i: (0, 0)),
            pl.BlockSpec((1, EMB), lambda i: (0, 0)),
        ],
        out_specs=(
            pl.BlockSpec((NB, EMB), lambda i: (i, 0)),
            pl.BlockSpec((NB, EMB), lambda i: (i, 0)),
        ),
        out_shape=(
            jax.ShapeDtypeStruct((N, EMB), jnp.float32),
            jax.ShapeDtypeStruct((N, EMB), jnp.float32),
        ),
    )(entity_embs, topic, pe[:, :, :N], q_emb, Wq, Ws, Wd, Wsml, b1row)


def _edge_mlp_kernel(rel_ref, g_ref, wr_ref, w2t_ref, b2_ref, out_ref):
    z = jnp.dot(rel_ref[...], wr_ref[...],
                preferred_element_type=jnp.float32) + g_ref[...]
    h = jnp.maximum(z, 0.0)
    b2v = b2_ref[...]
    out_ref[...] = (jnp.sum(h * w2t_ref[...], axis=1)
                    + b2v[0, 0]).reshape(1, EBLK)


def _edge_mlp(relation_embs, G, Wr, W2t, b2row, blk_off, nblk):
    ne = G.shape[0]
    return pl.pallas_call(
        _edge_mlp_kernel,
        grid=(nblk,),
        in_specs=[
            pl.BlockSpec((EBLK, EMB), lambda i: (i + blk_off, 0)),
            pl.BlockSpec((EBLK, EMB), lambda i: (i, 0)),
            pl.BlockSpec((EMB, EMB), lambda i: (0, 0)),
            pl.BlockSpec((1, EMB), lambda i: (0, 0)),
            pl.BlockSpec((1, 1), lambda i: (0, 0)),
        ],
        out_specs=pl.BlockSpec((1, EBLK), lambda i: (0, i)),
        out_shape=jax.ShapeDtypeStruct((1, ne), jnp.float32),
    )(relation_embs, G, Wr, W2t, b2row)


def kernel(edge_index, q_emb, entity_embs, relation_embs,
           topic_entity_one_hot, W1, b1, W2, b2):
    ei = edge_index.astype(jnp.int32)
    topic4 = jnp.concatenate(
        [topic_entity_one_hot,
         jnp.ones((N, 1), jnp.float32),
         jnp.zeros((N, 13), jnp.float32)], axis=1)
    topic4 = jnp.pad(topic4, ((0, NP - N), (0, 0)))
    zeros16 = jnp.zeros((NP, 16), jnp.float32)

    pe = _dde(ei, topic4, zeros16)

    Wsml = jnp.concatenate([W1[256:266], W1[522:532]], axis=0)
    Zs, Zd = _ztables(entity_embs, topic_entity_one_hot, pe, q_emb,
                      W1[0:128], W1[128:256], W1[394:522], Wsml,
                      b1.reshape(1, EMB))

    EH = ROWS_H * CHUNK
    G0 = _gather0(ei, Zs, Zd)
    G1 = _gather1(ei, Zs, Zd)

    Wr = W1[266:394]
    W2t = W2.reshape(1, EMB)
    b2row = b2.reshape(1, 1)
    nblk_h = EH // EBLK
    out0 = _edge_mlp(relation_embs, G0, Wr, W2t, b2row, 0, nblk_h)
    out1 = _edge_mlp(relation_embs, G1, Wr, W2t, b2row, nblk_h, nblk_h)
    return jnp.concatenate([out0, out1], axis=1).reshape(E, 1)


# speedup vs baseline: 1.2022x; 1.0379x over previous
import functools

import jax
import jax.numpy as jnp
from jax import lax
from jax.experimental import pallas as pl
from jax.experimental.pallas import tpu as pltpu
from jax.experimental.pallas import tpu_sc as plsc

N = 10000
NP = 10240
E = 160000
EMB = 128
CHUNK = 128
ROWS = E // CHUNK
NC = 2
NS = 16
NW = NC * NS
NPT = NP // NS
EBLK = 3200

DDE_BASE = ROWS // NS
DDE_EXTRA = ROWS - NS * DDE_BASE

_SC_PARAMS = pltpu.CompilerParams(use_tc_tiling_on_sc=False)


def _dde_body(ei_ref, topic4_ref, zeros16_ref, pe_ref,
              gidx_all, aidx_all, msg2, node_v, acc_sh, sem0, sem1):
    cid = lax.axis_index("c")
    sid = lax.axis_index("s")
    iota = lax.iota(jnp.int32, 16)
    mask01 = iota < 2
    constrow = jnp.where(iota == 2, 1.0, 0.0)

    start = (sid * DDE_BASE + jnp.minimum(sid, DDE_EXTRA)) * CHUNK
    cnt = DDE_BASE + jnp.where(sid < DDE_EXTRA, 1, 0)
    sems = (sem0, sem1)

    def chain(c):
        pltpu.sync_copy(ei_ref.at[c, pl.ds(start, DDE_BASE * CHUNK)],
                        gidx_all.at[pl.ds(0, DDE_BASE * CHUNK)])
        pltpu.sync_copy(ei_ref.at[1 - c, pl.ds(start, DDE_BASE * CHUNK)],
                        aidx_all.at[pl.ds(0, DDE_BASE * CHUNK)])

        @pl.when(sid < DDE_EXTRA)
        def _():
            pltpu.sync_copy(
                ei_ref.at[c, pl.ds(start + DDE_BASE * CHUNK, CHUNK)],
                gidx_all.at[pl.ds(DDE_BASE * CHUNK, CHUNK)])
            pltpu.sync_copy(
                ei_ref.at[1 - c, pl.ds(start + DDE_BASE * CHUNK, CHUNK)],
                aidx_all.at[pl.ds(DDE_BASE * CHUNK, CHUNK)])

        for r in range(2):
            pltpu.sync_copy(zeros16_ref.at[pl.ds(sid * NPT, NPT)],
                            acc_sh.at[pl.ds(sid * NPT, NPT)])
            plsc.subcore_barrier()

            table = topic4_ref if r == 0 else pe_ref.at[c, 0]

            def issue(j, s):
                pltpu.async_copy(
                    table.at[gidx_all.at[pl.ds(j * CHUNK, CHUNK)]],
                    msg2.at[s], sems[s])

            def drain(s):
                pltpu.make_async_copy(table.at[pl.ds(0, CHUNK)],
                                      msg2.at[s], sems[s]).wait()

            def scatter(j, s):
                pltpu.sync_copy(
                    msg2.at[s],
                    acc_sh.at[aidx_all.at[pl.ds(j * CHUNK, CHUNK)]],
                    add=True)

            issue(0, 0)
            issue(1, 1)

            def pair(j2, carry):
                a = 2 * j2
                drain(0)
                scatter(a, 0)

                @pl.when(a + 2 < cnt)
                def _():
                    issue(a + 2, 0)
                drain(1)
                scatter(a + 1, 1)

                @pl.when(a + 3 < cnt)
                def _():
                    issue(a + 3, 1)
                return carry
            lax.fori_loop(0, DDE_BASE // 2, pair, 0)

            @pl.when(sid < DDE_EXTRA)
            def _():
                drain(0)
                scatter(DDE_BASE, 0)

            plsc.subcore_barrier()

            pltpu.sync_copy(acc_sh.at[pl.ds(sid * NPT, NPT)], node_v)

            def dbody(i, carry):
                row = node_v[i, :]
                dvec = jnp.broadcast_to(row[2], (16,))
                inv = 1.0 / jnp.maximum(dvec, 1.0)
                node_v[i, :] = jnp.where(mask01, row * inv, constrow)
                return carry
            lax.fori_loop(0, NPT, dbody, 0)

            pltpu.sync_copy(node_v, pe_ref.at[c, r, pl.ds(sid * NPT, NPT)])
            plsc.subcore_barrier()

    @pl.when(cid == 0)
    def _():
        chain(0)

    @pl.when(cid == 1)
    def _():
        chain(1)


_dde = pl.kernel(
    _dde_body,
    out_type=jax.ShapeDtypeStruct((2, 2, NP, 16), jnp.float32),
    mesh=plsc.VectorSubcoreMesh(core_axis_name="c", subcore_axis_name="s"),
    scratch_types=[
        pltpu.VMEM(((DDE_BASE + 2) * CHUNK,), jnp.int32),
        pltpu.VMEM(((DDE_BASE + 2) * CHUNK,), jnp.int32),
        pltpu.VMEM((2, CHUNK, 16), jnp.float32),
        pltpu.VMEM((NPT, 16), jnp.float32),
        pltpu.VMEM_SHARED((NP, 16), jnp.float32),
        pltpu.SemaphoreType.DMA,
        pltpu.SemaphoreType.DMA,
    ],
    compiler_params=_SC_PARAMS,
)


def _make_gather(row_lo, nrows):
    base = nrows // NW
    extra = nrows - NW * base

    def body(ei_ref, zs_ref, zd_ref, g_ref,
             sidx_all, didx_all, abuf, bbuf, obuf,
             sa0, sa1, sb0, sb1, sw0, sw1):
        cid = lax.axis_index("c")
        sid = lax.axis_index("s")
        wid = sid * NC + cid
        rel_start = wid * base + jnp.minimum(wid, extra)
        start = (row_lo + rel_start) * CHUNK
        out_start = rel_start * CHUNK
        cnt = base + jnp.where(wid < extra, 1, 0)
        sas = (sa0, sa1)
        sbs = (sb0, sb1)
        sws = (sw0, sw1)

        pltpu.sync_copy(ei_ref.at[0, pl.ds(start, base * CHUNK)],
                        sidx_all.at[pl.ds(0, base * CHUNK)])
        pltpu.sync_copy(ei_ref.at[1, pl.ds(start, base * CHUNK)],
                        didx_all.at[pl.ds(0, base * CHUNK)])

        @pl.when(wid < extra)
        def _():
            pltpu.sync_copy(ei_ref.at[0, pl.ds(start + base * CHUNK, CHUNK)],
                            sidx_all.at[pl.ds(base * CHUNK, CHUNK)])
            pltpu.sync_copy(ei_ref.at[1, pl.ds(start + base * CHUNK, CHUNK)],
                            didx_all.at[pl.ds(base * CHUNK, CHUNK)])

        def issue(j, s):
            pltpu.async_copy(
                zs_ref.at[sidx_all.at[pl.ds(j * CHUNK, CHUNK)]],
                abuf.at[s], sas[s])
            pltpu.async_copy(
                zd_ref.at[didx_all.at[pl.ds(j * CHUNK, CHUNK)]],
                bbuf.at[s], sbs[s])

        def drain_g(s):
            pltpu.make_async_copy(zs_ref.at[pl.ds(0, CHUNK)], abuf.at[s],
                                  sas[s]).wait()
            pltpu.make_async_copy(zd_ref.at[pl.ds(0, CHUNK)], bbuf.at[s],
                                  sbs[s]).wait()

        def drain_w(s):
            pltpu.make_async_copy(obuf.at[s], g_ref.at[pl.ds(0, CHUNK)],
                                  sws[s]).wait()

        def process(j, s):
            drain_g(s)

            @pl.when(j >= 2)
            def _():
                drain_w(s)

            def add_row(i, c2):
                for k in range(EMB // 16):
                    sl = pl.ds(k * 16, 16)
                    obuf[s, i, sl] = abuf[s, i, sl] + bbuf[s, i, sl]
                return c2
            lax.fori_loop(0, CHUNK, add_row, 0)
            pltpu.async_copy(obuf.at[s],
                             g_ref.at[pl.ds(out_start + j * CHUNK, CHUNK)],
                             sws[s])

            @pl.when(j + 2 < cnt)
            def _():
                issue(j + 2, s)

        issue(0, 0)
        issue(1, 1)

        def pair(j2, carry):
            process(2 * j2, 0)
            process(2 * j2 + 1, 1)
            return carry
        lax.fori_loop(0, base // 2, pair, 0)

        if base % 2 == 1:
            process(base - 1, 0)

            @pl.when(wid < extra)
            def _():
                process(base, 1)
        else:
            @pl.when(wid < extra)
            def _():
                process(base, 0)

        drain_w(0)
        drain_w(1)

    return pl.kernel(
        body,
        out_type=jax.ShapeDtypeStruct((nrows * CHUNK, EMB), jnp.float32),
        mesh=plsc.VectorSubcoreMesh(core_axis_name="c", subcore_axis_name="s"),
        scratch_types=[
            pltpu.VMEM(((base + 3) * CHUNK,), jnp.int32),
            pltpu.VMEM(((base + 3) * CHUNK,), jnp.int32),
            pltpu.VMEM((2, CHUNK, EMB), jnp.float32),
            pltpu.VMEM((2, CHUNK, EMB), jnp.float32),
            pltpu.VMEM((2, CHUNK, EMB), jnp.float32),
            pltpu.SemaphoreType.DMA,
            pltpu.SemaphoreType.DMA,
            pltpu.SemaphoreType.DMA,
            pltpu.SemaphoreType.DMA,
            pltpu.SemaphoreType.DMA,
            pltpu.SemaphoreType.DMA,
        ],
    )


ROWS_H = ROWS // 2
_gather0 = _make_gather(0, ROWS_H)
_gather1 = _make_gather(ROWS_H, ROWS - ROWS_H)


def _ztables_kernel(ent_ref, topic_ref, pe_ref, q_ref, wq_ref,
                    ws_ref, wd_ref, wsml_ref, b1_ref, zs_ref, zd_ref):
    qc = jnp.dot(q_ref[...], wq_ref[...],
                 preferred_element_type=jnp.float32) + b1_ref[...]
    ent = ent_ref[...]
    zs = jnp.dot(ent, ws_ref[...], preferred_element_type=jnp.float32) + qc
    zd = jnp.dot(ent, wd_ref[...], preferred_element_type=jnp.float32)

    feats = [topic_ref[:, 0:1], topic_ref[:, 1:2]]
    for c in range(2):
        for r in range(2):
            pe = pe_ref[c, r]
            feats.append(pe[:, 0:1])
            feats.append(pe[:, 1:2])
    for k, f in enumerate(feats):
        zs = zs + f * wsml_ref[k, :][None, :]
        zd = zd + f * wsml_ref[10 + k, :][None, :]
    zs_ref[...] = zs
    zd_ref[...] = zd


def _ztables(entity_embs, topic, pe, q_emb, Wq, Ws, Wd, Wsml, b1row):
    NB = 1000
    return pl.pallas_call(
        _ztables_kernel,
        grid=(N // NB,),
        in_specs=[
            pl.BlockSpec((NB, EMB), lambda i: (i, 0)),
            pl.BlockSpec((NB, 2), lambda i: (i, 0)),
            pl.BlockSpec((2, 2, NB, 16), lambda i: (0, 0, i, 0)),
            pl.BlockSpec((1, EMB), lambda i: (0, 0)),
            pl.BlockSpec((EMB, EMB), lambda i: (0, 0)),
            pl.BlockSpec((EMB, EMB), lambda i: (0, 0)),
            pl.BlockSpec((EMB, EMB), lambda i: (0, 0)),
            pl.BlockSpec((20, EMB), lambda i: (0, 0)),
            pl.BlockSpec((1, EMB), lambda i: (0, 0)),
        ],
        out_specs=(
            pl.BlockSpec((NB, EMB), lambda i: (i, 0)),
            pl.BlockSpec((NB, EMB), lambda i: (i, 0)),
        ),
        out_shape=(
            jax.ShapeDtypeStruct((N, EMB), jnp.float32),
            jax.ShapeDtypeStruct((N, EMB), jnp.float32),
        ),
    )(entity_embs, topic, pe[:, :, :N], q_emb, Wq, Ws, Wd, Wsml, b1row)


def _edge_mlp_kernel(rel_ref, g_ref, wr_ref, w2t_ref, b2_ref, out_ref):
    z = jnp.dot(rel_ref[...], wr_ref[...],
                preferred_element_type=jnp.float32) + g_ref[...]
    h = jnp.maximum(z, 0.0)
    b2v = b2_ref[...]
    out_ref[...] = lax.dot_general(
        w2t_ref[...], h, (((1,), (1,)), ((), ())),
        preferred_element_type=jnp.float32) + b2v[0, 0]


def _edge_mlp(relation_embs, G, Wr, W2t, b2row, blk_off, nblk):
    ne = G.shape[0]
    return pl.pallas_call(
        _edge_mlp_kernel,
        grid=(nblk,),
        in_specs=[
            pl.BlockSpec((EBLK, EMB), lambda i: (i + blk_off, 0)),
            pl.BlockSpec((EBLK, EMB), lambda i: (i, 0)),
            pl.BlockSpec((EMB, EMB), lambda i: (0, 0)),
            pl.BlockSpec((1, EMB), lambda i: (0, 0)),
            pl.BlockSpec((1, 1), lambda i: (0, 0)),
        ],
        out_specs=pl.BlockSpec((1, EBLK), lambda i: (0, i)),
        out_shape=jax.ShapeDtypeStruct((1, ne), jnp.float32),
    )(relation_embs, G, Wr, W2t, b2row)


def kernel(edge_index, q_emb, entity_embs, relation_embs,
           topic_entity_one_hot, W1, b1, W2, b2):
    ei = edge_index.astype(jnp.int32)
    topic4 = jnp.concatenate(
        [topic_entity_one_hot,
         jnp.ones((N, 1), jnp.float32),
         jnp.zeros((N, 13), jnp.float32)], axis=1)
    topic4 = jnp.pad(topic4, ((0, NP - N), (0, 0)))
    zeros16 = jnp.zeros((NP, 16), jnp.float32)

    pe = _dde(ei, topic4, zeros16)

    Wsml = jnp.concatenate([W1[256:266], W1[522:532]], axis=0)
    Zs, Zd = _ztables(entity_embs, topic_entity_one_hot, pe, q_emb,
                      W1[0:128], W1[128:256], W1[394:522], Wsml,
                      b1.reshape(1, EMB))

    EH = ROWS_H * CHUNK
    G0 = _gather0(ei, Zs, Zd)
    G1 = _gather1(ei, Zs, Zd)

    Wr = W1[266:394]
    W2t = W2.reshape(1, EMB)
    b2row = b2.reshape(1, 1)
    nblk_h = EH // EBLK
    out0 = _edge_mlp(relation_embs, G0, Wr, W2t, b2row, 0, nblk_h)
    out1 = _edge_mlp(relation_embs, G1, Wr, W2t, b2row, nblk_h, nblk_h)
    return jnp.concatenate([out0, out1], axis=1).reshape(E, 1)
